# baseline (speedup 1.0000x reference)
import jax
import jax.numpy as jnp
from jax import lax
from jax.experimental import pallas as pl
from jax.experimental.pallas import tpu as pltpu
from jax.experimental.pallas import tpu_sc as plsc

_EMBED_DIM = 1152
_NUM_CORES = 2
_NUM_SUBCORES = 16
_NUM_WORKERS = _NUM_CORES * _NUM_SUBCORES
_T_MAIN = 48
_T_PAD = 56


def _gather_body(idx_hbm, table_hbm, out_hbm, tail_hbm,
                 idx0, idx1, rows0, rows1,
                 isem0, isem1, gsem0, gsem1, ssem0, ssem1, tsem0, tsem1):
    wid = lax.axis_index("s") * _NUM_CORES + lax.axis_index("c")
    n_b = idx_hbm.shape[0] // _NUM_WORKERS
    base = wid * n_b
    idxs = (idx0, idx1)
    bufs = (rows0, rows1)
    isems = (isem0, isem1)
    gsems = (gsem0, gsem1)
    ssems = (ssem0, ssem1)
    tsems = (tsem0, tsem1)

    def start_gather(slot):
        pltpu.async_copy(table_hbm.at[idxs[slot]], bufs[slot], gsems[slot])

    def wait_gather(slot):
        pltpu.make_async_copy(
            table_hbm.at[idxs[slot]], bufs[slot], gsems[slot]).wait()

    def start_stores(r, slot):
        pltpu.async_copy(
            bufs[slot].at[pl.ds(0, _T_MAIN)],
            out_hbm.at[base + r, pl.ds(0, _T_MAIN)], ssems[slot])
        pltpu.async_copy(
            bufs[slot].at[pl.ds(_T_MAIN, 8)],
            tail_hbm.at[base + r], tsems[slot])

    def wait_stores(r, slot):
        pltpu.make_async_copy(
            bufs[slot].at[pl.ds(0, _T_MAIN)],
            out_hbm.at[base + r, pl.ds(0, _T_MAIN)], ssems[slot]).wait()
        pltpu.make_async_copy(
            bufs[slot].at[pl.ds(_T_MAIN, 8)],
            tail_hbm.at[base + r], tsems[slot]).wait()

    pltpu.async_copy(idx_hbm.at[base], idx0, isem0)
    pltpu.async_copy(idx_hbm.at[base + 1], idx1, isem1)
    pltpu.make_async_copy(idx_hbm.at[base], idx0, isem0).wait()
    start_gather(0)

    def body(i, carry):
        for s in range(2):
            g = 2 * i + s
            ns = 1 - s
            wait_gather(s)

            @pl.when(g + 2 < n_b)
            def _():
                pltpu.async_copy(idx_hbm.at[base + g + 2], idxs[s], isems[s])

            start_stores(g, s)

            @pl.when(g > 0)
            def _():
                wait_stores(g - 1, ns)

            @pl.when(g + 1 < n_b)
            def _():
                pltpu.make_async_copy(
                    idx_hbm.at[base + g + 1], idxs[ns], isems[ns]).wait()
                start_gather(ns)
        return carry

    lax.fori_loop(0, n_b // 2, body, 0)
    wait_stores(n_b - 1, 1)


def _splice_body(tail_ref, main_ref, out_ref, sem):
    del main_ref
    n = tail_ref.shape[0]
    blk = n // 32
    for k in range(32):
        pltpu.async_copy(
            tail_ref.at[pl.ds(k * blk, blk), pl.ds(0, 2), :],
            out_ref.at[pl.ds(k * blk, blk), pl.ds(_T_MAIN, 2), :],
            sem)
    for k in range(32):
        pltpu.make_async_copy(
            tail_ref.at[pl.ds(k * blk, blk), pl.ds(0, 2), :],
            out_ref.at[pl.ds(k * blk, blk), pl.ds(_T_MAIN, 2), :],
            sem).wait()


@jax.jit
def kernel(input, weight):
    b, t = input.shape
    n_pad = _T_PAD - t
    pad_idx = (jnp.arange(b, dtype=jnp.int32)[:, None] * n_pad
               + jnp.arange(n_pad, dtype=jnp.int32)[None, :]) % weight.shape[0]
    idx = jnp.concatenate([input.astype(jnp.int32), pad_idx], axis=1)
    mesh = plsc.VectorSubcoreMesh(core_axis_name="c", subcore_axis_name="s")
    out_main, out_tail = pl.kernel(
        _gather_body,
        out_type=(
            jax.ShapeDtypeStruct((b, t, _EMBED_DIM), jnp.float32),
            jax.ShapeDtypeStruct((b, 8, _EMBED_DIM), jnp.float32),
        ),
        mesh=mesh,
        scratch_types=[
            pltpu.VMEM((_T_PAD,), jnp.int32),
            pltpu.VMEM((_T_PAD,), jnp.int32),
            pltpu.VMEM((_T_PAD, _EMBED_DIM), jnp.float32),
            pltpu.VMEM((_T_PAD, _EMBED_DIM), jnp.float32),
            pltpu.SemaphoreType.DMA,
            pltpu.SemaphoreType.DMA,
            pltpu.SemaphoreType.DMA,
            pltpu.SemaphoreType.DMA,
            pltpu.SemaphoreType.DMA,
            pltpu.SemaphoreType.DMA,
            pltpu.SemaphoreType.DMA,
            pltpu.SemaphoreType.DMA,
        ],
    )(idx, weight)

    out = pl.pallas_call(
        _splice_body,
        in_specs=[
            pl.BlockSpec(memory_space=pltpu.HBM),
            pl.BlockSpec(memory_space=pltpu.HBM),
        ],
        out_specs=pl.BlockSpec(memory_space=pltpu.HBM),
        out_shape=jax.ShapeDtypeStruct((b, t, _EMBED_DIM), jnp.float32),
        scratch_shapes=[pltpu.SemaphoreType.DMA],
        input_output_aliases={1: 0},
    )(out_tail, out_main)
    return out


# speedup vs baseline: 1.7274x; 1.7274x over previous
import jax
import jax.numpy as jnp
from jax import lax
from jax.experimental import pallas as pl
from jax.experimental.pallas import tpu as pltpu
from jax.experimental.pallas import tpu_sc as plsc

_EMBED_DIM = 1152
_NUM_CORES = 2
_NUM_SUBCORES = 16
_NUM_WORKERS = _NUM_CORES * _NUM_SUBCORES
_T_MAIN = 48
_T_PAD = 56


def _gather_body(idx_hbm, table_hbm, out_hbm, tail_hbm,
                 idx0, idx1, rows0, rows1,
                 isem0, isem1, gsem0, gsem1, ssem0, ssem1, tsem0, tsem1):
    wid = lax.axis_index("s") * _NUM_CORES + lax.axis_index("c")
    n_b = idx_hbm.shape[0] // _NUM_WORKERS
    base = wid * n_b
    idxs = (idx0, idx1)
    bufs = (rows0, rows1)
    isems = (isem0, isem1)
    gsems = (gsem0, gsem1)
    ssems = (ssem0, ssem1)
    tsems = (tsem0, tsem1)

    def start_gather(slot):
        pltpu.async_copy(table_hbm.at[idxs[slot]], bufs[slot], gsems[slot])

    def wait_gather(slot):
        pltpu.make_async_copy(
            table_hbm.at[idxs[slot]], bufs[slot], gsems[slot]).wait()

    def start_stores(r, slot):
        pltpu.async_copy(
            bufs[slot].at[pl.ds(0, _T_MAIN)],
            out_hbm.at[base + r, pl.ds(0, _T_MAIN)], ssems[slot])
        pltpu.async_copy(
            bufs[slot].at[pl.ds(_T_MAIN, 8)],
            tail_hbm.at[base + r], tsems[slot])

    def wait_stores(r, slot):
        pltpu.make_async_copy(
            bufs[slot].at[pl.ds(0, _T_MAIN)],
            out_hbm.at[base + r, pl.ds(0, _T_MAIN)], ssems[slot]).wait()
        pltpu.make_async_copy(
            bufs[slot].at[pl.ds(_T_MAIN, 8)],
            tail_hbm.at[base + r], tsems[slot]).wait()

    pltpu.async_copy(idx_hbm.at[base], idx0, isem0)
    pltpu.async_copy(idx_hbm.at[base + 1], idx1, isem1)
    pltpu.make_async_copy(idx_hbm.at[base], idx0, isem0).wait()
    start_gather(0)

    def body(i, carry):
        for s in range(2):
            g = 2 * i + s
            ns = 1 - s
            wait_gather(s)

            @pl.when(g + 2 < n_b)
            def _():
                pltpu.async_copy(idx_hbm.at[base + g + 2], idxs[s], isems[s])

            start_stores(g, s)

            @pl.when(g > 0)
            def _():
                wait_stores(g - 1, ns)

            @pl.when(g + 1 < n_b)
            def _():
                pltpu.make_async_copy(
                    idx_hbm.at[base + g + 1], idxs[ns], isems[ns]).wait()
                start_gather(ns)
        return carry

    lax.fori_loop(0, n_b // 2, body, 0)
    wait_stores(n_b - 1, 1)


def _splice_body(tail_ref, main_ref, out_ref, sem):
    del main_ref
    n = tail_ref.shape[0]
    blk = n // 32
    for k in range(32):
        pltpu.async_copy(
            tail_ref.at[pl.ds(k * blk, blk), pl.ds(0, 2), :],
            out_ref.at[pl.ds(k * blk, blk), pl.ds(_T_MAIN, 2), :],
            sem)
    for k in range(32):
        pltpu.make_async_copy(
            tail_ref.at[pl.ds(k * blk, blk), pl.ds(0, 2), :],
            out_ref.at[pl.ds(k * blk, blk), pl.ds(_T_MAIN, 2), :],
            sem).wait()


@jax.jit
def kernel(input, weight):
    b, t = input.shape
    n_pad = _T_PAD - t
    pad_idx = (jnp.arange(b, dtype=jnp.int32)[:, None] * n_pad
               + jnp.arange(n_pad, dtype=jnp.int32)[None, :]) % weight.shape[0]
    idx = jnp.concatenate([input.astype(jnp.int32), pad_idx], axis=1)
    mesh = plsc.VectorSubcoreMesh(core_axis_name="c", subcore_axis_name="s")
    out_main, out_tail = pl.kernel(
        _gather_body,
        out_type=(
            jax.ShapeDtypeStruct((b, t, _EMBED_DIM), jnp.float32),
            jax.ShapeDtypeStruct((b, 8, _EMBED_DIM), jnp.float32),
        ),
        mesh=mesh,
        scratch_types=[
            pltpu.VMEM((_T_PAD,), jnp.int32),
            pltpu.VMEM((_T_PAD,), jnp.int32),
            pltpu.VMEM((_T_PAD, _EMBED_DIM), jnp.float32),
            pltpu.VMEM((_T_PAD, _EMBED_DIM), jnp.float32),
            pltpu.SemaphoreType.DMA,
            pltpu.SemaphoreType.DMA,
            pltpu.SemaphoreType.DMA,
            pltpu.SemaphoreType.DMA,
            pltpu.SemaphoreType.DMA,
            pltpu.SemaphoreType.DMA,
            pltpu.SemaphoreType.DMA,
            pltpu.SemaphoreType.DMA,
        ],
    )(idx, weight)

    return lax.dynamic_update_slice(
        out_main, out_tail[:, :2, :], (0, _T_MAIN, 0))
